# baseline (speedup 1.0000x reference)
import functools

import jax
import jax.numpy as jnp
from jax import lax
from jax.experimental import pallas as pl
from jax.experimental.pallas import tpu as pltpu
from jax.experimental.pallas import tpu_sc as plsc

_INFO = plsc.get_sparse_core_info()
_NC, _NS = _INFO.num_cores, _INFO.num_subcores
_NW = _NC * _NS
_IDX_CHUNK = 128


_MLP_BLK = 1000


def _mlp_pipe_body(pe_hbm, w1_ref, b1_ref, w2_ref, b2_ref, out_hbm, *scratch):
    xbufs = scratch[0:2]
    obufs = scratch[2:4]
    isems = scratch[4:6]
    osems = scratch[6:8]
    n_blk = pe_hbm.shape[0] // _MLP_BLK
    w1 = w1_ref[...]
    b1 = b1_ref[...]
    w2 = w2_ref[...]
    b2 = b2_ref[...]

    def in_copy(i):
        return pltpu.make_async_copy(
            pe_hbm.at[pl.ds(i * _MLP_BLK, _MLP_BLK)], xbufs[i % 2], isems[i % 2]
        )

    def out_copy(i):
        return pltpu.make_async_copy(
            obufs[i % 2], out_hbm.at[pl.ds(i * _MLP_BLK, _MLP_BLK)], osems[i % 2]
        )

    in_copy(0).start()
    for i in range(n_blk):
        if i + 1 < n_blk:
            in_copy(i + 1).start()
        in_copy(i).wait()
        x = xbufs[i % 2][...]
        h = jnp.dot(x, w1, preferred_element_type=jnp.float32) + b1
        h = h * jax.nn.sigmoid(h)
        o = jnp.dot(h, w2, preferred_element_type=jnp.float32) + b2
        if i >= 2:
            out_copy(i - 2).wait()
        obufs[i % 2][...] = o
        out_copy(i).start()
    for i in range(max(0, n_blk - 2), n_blk):
        out_copy(i).wait()


def _mlp_table(pe2d, W1, b1, W2, b2):
    v, h = pe2d.shape
    return pl.pallas_call(
        _mlp_pipe_body,
        in_specs=[
            pl.BlockSpec(memory_space=pl.ANY),
            pl.BlockSpec((h, h), lambda: (0, 0)),
            pl.BlockSpec((1, h), lambda: (0, 0)),
            pl.BlockSpec((h, h), lambda: (0, 0)),
            pl.BlockSpec((1, h), lambda: (0, 0)),
        ],
        out_specs=pl.BlockSpec(memory_space=pl.ANY),
        out_shape=jax.ShapeDtypeStruct((v, h), jnp.float32),
        scratch_shapes=(
            [pltpu.VMEM((_MLP_BLK, 128), jnp.float32)] * 4
            + [pltpu.SemaphoreType.DMA] * 4
        ),
    )(pe2d, W1, b1.reshape(1, h), W2, b2.reshape(1, h))


def _make_gather(V, D, B):
    b_per_w = B // _NW
    n_chunks = b_per_w // _IDX_CHUNK
    mesh = plsc.VectorSubcoreMesh(core_axis_name="c", subcore_axis_name="s")

    @functools.partial(
        pl.kernel,
        mesh=mesh,
        out_type=jax.ShapeDtypeStruct((B, D), jnp.float32),
        scratch_types=[
            pltpu.VMEM((n_chunks, _IDX_CHUNK), jnp.int32),
            pltpu.VMEM((b_per_w, D), jnp.float32),
        ]
        + [pltpu.SemaphoreType.DMA] * (n_chunks + 1),
    )
    def gather_k(table_hbm, idx_hbm, out_hbm, idx_v, rows_v, *sems):
        gsems, wsem = sems[:n_chunks], sems[n_chunks]
        wid = lax.axis_index("s") * _NC + lax.axis_index("c")
        base = wid * b_per_w
        pltpu.sync_copy(idx_hbm.at[wid], idx_v)
        gathers = []
        for j in range(n_chunks):
            gathers.append(
                pltpu.async_copy(
                    table_hbm.at[idx_v.at[j]],
                    rows_v.at[pl.ds(j * _IDX_CHUNK, _IDX_CHUNK)],
                    gsems[j],
                )
            )
        writes = []
        for j in range(n_chunks):
            gathers[j].wait()
            writes.append(
                pltpu.async_copy(
                    rows_v.at[pl.ds(j * _IDX_CHUNK, _IDX_CHUNK)],
                    out_hbm.at[pl.ds(base + j * _IDX_CHUNK, _IDX_CHUNK)],
                    wsem,
                )
            )
        for w in writes:
            w.wait()

    return gather_k


def kernel(timesteps, pe, W1, b1, W2, b2):
    B = timesteps.shape[0]
    V, H = pe.shape[0], pe.shape[-1]
    pe2d = pe.reshape(V, H)
    table = _mlp_table(pe2d, W1, b1, W2, b2)
    idx = timesteps.astype(jnp.int32).reshape(_NW, (B // _NW) // _IDX_CHUNK, _IDX_CHUNK)
    out = _make_gather(V, H, B)(table, idx)
    return out.reshape(1, B, H)


# speedup vs baseline: 1.0609x; 1.0609x over previous
import functools

import jax
import jax.numpy as jnp
from jax import lax
from jax.experimental import pallas as pl
from jax.experimental.pallas import tpu as pltpu
from jax.experimental.pallas import tpu_sc as plsc

_INFO = plsc.get_sparse_core_info()
_NC, _NS = _INFO.num_cores, _INFO.num_subcores
_NW = _NC * _NS
_IDX_CHUNK = 128


def _mlp_body(pe_ref, w1_ref, b1_ref, w2_ref, b2_ref, out_ref):
    x = pe_ref[...]
    h = jnp.dot(x, w1_ref[...], preferred_element_type=jnp.float32)
    h = h + b1_ref[...]
    h = h * jax.nn.sigmoid(h)
    o = jnp.dot(h, w2_ref[...], preferred_element_type=jnp.float32)
    out_ref[...] = o + b2_ref[...]


def _mlp_table(pe2d, W1, b1, W2, b2):
    v, h = pe2d.shape
    return pl.pallas_call(
        _mlp_body,
        out_shape=jax.ShapeDtypeStruct((v, h), jnp.float32),
    )(pe2d, W1, b1.reshape(1, h), W2, b2.reshape(1, h))


def _make_gather(V, D, B):
    b_per_w = B // _NW
    n_chunks = b_per_w // _IDX_CHUNK
    mesh = plsc.VectorSubcoreMesh(core_axis_name="c", subcore_axis_name="s")

    @functools.partial(
        pl.kernel,
        mesh=mesh,
        out_type=jax.ShapeDtypeStruct((B, D), jnp.float32),
        scratch_types=[
            pltpu.VMEM((n_chunks, _IDX_CHUNK), jnp.int32),
            pltpu.VMEM((b_per_w, D), jnp.float32),
        ]
        + [pltpu.SemaphoreType.DMA] * (n_chunks + 1),
    )
    def gather_k(table_hbm, idx_hbm, out_hbm, idx_v, rows_v, *sems):
        gsems, wsem = sems[:n_chunks], sems[n_chunks]
        wid = lax.axis_index("s") * _NC + lax.axis_index("c")
        base = wid * b_per_w
        pltpu.sync_copy(idx_hbm.at[wid], idx_v)
        gathers = []
        for j in range(n_chunks):
            gathers.append(
                pltpu.async_copy(
                    table_hbm.at[idx_v.at[j]],
                    rows_v.at[pl.ds(j * _IDX_CHUNK, _IDX_CHUNK)],
                    gsems[j],
                )
            )
        writes = []
        for j in range(n_chunks):
            gathers[j].wait()
            writes.append(
                pltpu.async_copy(
                    rows_v.at[pl.ds(j * _IDX_CHUNK, _IDX_CHUNK)],
                    out_hbm.at[pl.ds(base + j * _IDX_CHUNK, _IDX_CHUNK)],
                    wsem,
                )
            )
        for w in writes:
            w.wait()

    return gather_k


def kernel(timesteps, pe, W1, b1, W2, b2):
    B = timesteps.shape[0]
    V, H = pe.shape[0], pe.shape[-1]
    pe2d = pe.reshape(V, H)
    table = _mlp_table(pe2d, W1, b1, W2, b2)
    idx = timesteps.astype(jnp.int32).reshape(_NW, (B // _NW) // _IDX_CHUNK, _IDX_CHUNK)
    out = _make_gather(V, H, B)(table, idx)
    return out.reshape(1, B, H)
